# Initial kernel scaffold; baseline (speedup 1.0000x reference)
import jax
import jax.numpy as jnp
from jax.experimental import pallas as pl


def kernel(x, edge_index, W1, b1, W2, b2, temp, Wf, bf):
    raise NotImplementedError("write your pallas kernel here")


# speedup vs baseline: 61.8379x; 61.8379x over previous
import functools

import jax
import jax.numpy as jnp
from jax import lax
from jax.experimental import pallas as pl
from jax.experimental.pallas import tpu as pltpu
from jax.experimental.pallas import tpu_sc as plsc

_NT = 16
_CH = 2000


def _mlp_kernel(x_ref, w1_ref, b1_ref, w2_ref, b2_ref, wf_ref, g_ref):
    xb = x_ref[...]
    h1 = jnp.maximum(xb * w1_ref[...] + b1_ref[...][None, :], 0.0)
    h2 = jnp.dot(h1, w2_ref[...], preferred_element_type=jnp.float32)
    h2 = jnp.maximum(h2 + b2_ref[...][None, :], 0.0)
    g_ref[...] = jnp.sum(h2 * wf_ref[...][:, 0][None, :], axis=1,
                         keepdims=True)


def _run_mlp(x_pad, W1, b1, W2, b2, Wf, n_pad):
    blk = n_pad // _NT
    full = lambda shape: pl.BlockSpec(shape, lambda i: (0,) * len(shape))
    return pl.pallas_call(
        _mlp_kernel,
        grid=(_NT,),
        in_specs=[
            pl.BlockSpec((blk, 1), lambda i: (i, 0)),
            full((1, 32)), full((32,)), full((32, 64)), full((64,)),
            full((64, 1)),
        ],
        out_specs=pl.BlockSpec((blk, 1), lambda i: (i, 0)),
        out_shape=jax.ShapeDtypeStruct((n_pad, 1), jnp.float32),
    )(x_pad, W1, b1, W2, b2, Wf)


def _newton_rsqrt(d):
    i = lax.bitcast_convert_type(d, jnp.int32)
    i = jnp.int32(0x5F3759DF) - lax.shift_right_arithmetic(i, 1)
    y = lax.bitcast_convert_type(i, jnp.float32)
    for _ in range(3):
        y = y * (1.5 - 0.5 * d * y * y)
    return y


def _make_propagate(n_pad, e_pad, k_hops):
    slc = n_pad // _NT
    ec = e_pad // _NT
    nch = ec // _CH
    nv = slc // 16

    mesh = plsc.VectorSubcoreMesh(core_axis_name="c", subcore_axis_name="s",
                                  num_cores=1)

    def body(src_hbm, dst_hbm, g_hbm, consts_hbm, out_hbm,
             p_sh, acc, p_loc, dinv_loc, out_loc, q_loc,
             src_buf, dst_buf, val_buf, c_loc):
        s = lax.axis_index("s")
        base_n = s * slc
        base_e = s * ec
        nsl = pl.ds(base_n, slc)

        pltpu.sync_copy(consts_hbm, c_loc)

        def zfill(i, _):
            q_loc[pl.ds(i * 16, 16)] = jnp.zeros((16,), jnp.float32)
            return _
        lax.fori_loop(0, nv, zfill, None)
        pltpu.sync_copy(q_loc, acc.at[nsl])

        def ofill(i, _):
            val_buf[pl.ds(i * 16, 16)] = jnp.full((16,), 1.0, jnp.float32)
            return _
        lax.fori_loop(0, _CH // 16, ofill, None)
        plsc.subcore_barrier()

        def deg_body(c, _):
            e0 = base_e + c * _CH
            pltpu.sync_copy(dst_hbm.at[pl.ds(e0, _CH)], dst_buf)
            pltpu.sync_copy(val_buf, acc.at[dst_buf], add=True)
            return _
        lax.fori_loop(0, nch, deg_body, None)
        plsc.subcore_barrier()

        pltpu.sync_copy(acc.at[nsl], q_loc)
        pltpu.sync_copy(g_hbm.at[nsl], p_loc)
        t0 = c_loc[pl.ds(0, 16)]
        bfv = c_loc[pl.ds((k_hops + 1) * 16, 16)]

        def init_body(i, _):
            sl = pl.ds(i * 16, 16)
            dv = _newton_rsqrt(q_loc[sl] + 1.0)
            dinv_loc[sl] = dv
            gv = p_loc[sl]
            out_loc[sl] = t0 * gv + bfv
            p_loc[sl] = dv * gv
            return _
        lax.fori_loop(0, nv, init_body, None)

        pltpu.sync_copy(p_loc, p_sh.at[nsl])
        pltpu.sync_copy(p_loc, acc.at[nsl])
        plsc.subcore_barrier()

        for k in range(1, k_hops + 1):
            def edge_body(c, _):
                e0 = base_e + c * _CH
                pltpu.sync_copy(src_hbm.at[pl.ds(e0, _CH)], src_buf)
                pltpu.sync_copy(dst_hbm.at[pl.ds(e0, _CH)], dst_buf)
                pltpu.sync_copy(p_sh.at[src_buf], val_buf)
                pltpu.sync_copy(val_buf, acc.at[dst_buf], add=True)
                return _
            lax.fori_loop(0, nch, edge_body, None)
            plsc.subcore_barrier()

            pltpu.sync_copy(acc.at[nsl], q_loc)
            tk = c_loc[pl.ds(k * 16, 16)]

            def ew_body(i, _):
                sl = pl.ds(i * 16, 16)
                dv = dinv_loc[sl]
                y = dv * q_loc[sl]
                out_loc[sl] = out_loc[sl] + tk * y
                p_loc[sl] = dv * y
                return _
            lax.fori_loop(0, nv, ew_body, None)

            if k < k_hops:
                pltpu.sync_copy(p_loc, p_sh.at[nsl])
                pltpu.sync_copy(p_loc, acc.at[nsl])
                plsc.subcore_barrier()

        pltpu.sync_copy(out_loc, out_hbm.at[nsl])

    return pl.kernel(
        body,
        out_type=jax.ShapeDtypeStruct((n_pad,), jnp.float32),
        mesh=mesh,
        scratch_types=[
            pltpu.VMEM_SHARED((n_pad,), jnp.float32),
            pltpu.VMEM_SHARED((n_pad,), jnp.float32),
            pltpu.VMEM((slc,), jnp.float32),
            pltpu.VMEM((slc,), jnp.float32),
            pltpu.VMEM((slc,), jnp.float32),
            pltpu.VMEM((slc,), jnp.float32),
            pltpu.VMEM((_CH,), jnp.int32),
            pltpu.VMEM((_CH,), jnp.int32),
            pltpu.VMEM((_CH,), jnp.float32),
            pltpu.VMEM((16 * (k_hops + 2),), jnp.float32),
        ],
    )


def kernel(x, edge_index, W1, b1, W2, b2, temp, Wf, bf):
    n = x.shape[0]
    e = edge_index.shape[1]
    k_hops = temp.shape[0] - 1

    n_pad = ((n + _NT * 16 - 1) // (_NT * 16)) * (_NT * 16)
    e_pad = ((e + _NT * _CH - 1) // (_NT * _CH)) * (_NT * _CH)

    x_pad = jnp.pad(x, ((0, n_pad - n), (0, 0)))
    g = _run_mlp(x_pad, W1, b1, W2, b2, Wf, n_pad)[:, 0]

    src = jnp.pad(edge_index[0], (0, e_pad - e), constant_values=n_pad - 1)
    dst = jnp.pad(edge_index[1], (0, e_pad - e), constant_values=n_pad - 1)

    consts = jnp.concatenate([jnp.repeat(temp, 16), jnp.repeat(bf, 16)])

    prop = _make_propagate(n_pad, e_pad, k_hops)
    out = prop(src, dst, g, consts)
    return out[:n, None]


# speedup vs baseline: 116.4961x; 1.8839x over previous
import jax
import jax.numpy as jnp
from jax import lax
from jax.experimental import pallas as pl
from jax.experimental.pallas import tpu as pltpu
from jax.experimental.pallas import tpu_sc as plsc

_NT = 16
_CH = 5000


def _mlp_kernel(x_ref, w1_ref, b1_ref, w2_ref, b2_ref, wf_ref, g_ref):
    xb = x_ref[...]
    h1 = jnp.maximum(xb * w1_ref[...] + b1_ref[...][None, :], 0.0)
    h2 = jnp.dot(h1, w2_ref[...], preferred_element_type=jnp.float32)
    h2 = jnp.maximum(h2 + b2_ref[...][None, :], 0.0)
    g_ref[...] = jnp.sum(h2 * wf_ref[...][:, 0][None, :], axis=1,
                         keepdims=True)


def _run_mlp(x_pad, W1, b1, W2, b2, Wf, n_pad):
    blk = n_pad // _NT
    full = lambda shape: pl.BlockSpec(shape, lambda i: (0,) * len(shape))
    return pl.pallas_call(
        _mlp_kernel,
        grid=(_NT,),
        in_specs=[
            pl.BlockSpec((blk, 1), lambda i: (i, 0)),
            full((1, 32)), full((32,)), full((32, 64)), full((64,)),
            full((64, 1)),
        ],
        out_specs=pl.BlockSpec((blk, 1), lambda i: (i, 0)),
        out_shape=jax.ShapeDtypeStruct((n_pad, 1), jnp.float32),
    )(x_pad, W1, b1, W2, b2, Wf)


def _newton_rsqrt(d):
    i = lax.bitcast_convert_type(d, jnp.int32)
    i = jnp.int32(0x5F3759DF) - lax.shift_right_arithmetic(i, 1)
    y = lax.bitcast_convert_type(i, jnp.float32)
    for _ in range(3):
        y = y * (1.5 - 0.5 * d * y * y)
    return y


def _make_propagate(n_pad, e_pad, k_hops):
    slc = n_pad // _NT
    ec = e_pad // _NT
    nch = ec // _CH
    nv = slc // 16

    mesh = plsc.VectorSubcoreMesh(core_axis_name="c", subcore_axis_name="s",
                                  num_cores=1)

    def body(src_hbm, dst_hbm, g_hbm, consts_hbm, out_hbm,
             p_sh, acc, p_loc, dinv_loc, out_loc, q_loc,
             sb0, sb1, sb2, sb3, db0, db1, db2, db3,
             vb0, vb1, vb2, vb3, c_loc,
             ld0, ld1, ld2, ld3, sc0, sc1, sc2, sc3):
        sbufs = (sb0, sb1, sb2, sb3)
        dbufs = (db0, db1, db2, db3)
        vbufs = (vb0, vb1, vb2, vb3)
        lds = (ld0, ld1, ld2, ld3)
        scs = (sc0, sc1, sc2, sc3)

        s = lax.axis_index("s")
        base_n = s * slc
        base_e = s * ec
        nsl = pl.ds(base_n, slc)

        def issue_pref(chunk_idx, t):
            e0 = base_e + chunk_idx * _CH
            pltpu.async_copy(src_hbm.at[pl.ds(e0, _CH)], sbufs[t], lds[t])
            pltpu.async_copy(dst_hbm.at[pl.ds(e0, _CH)], dbufs[t], lds[t])

        def wait_ld(t):
            pltpu.make_async_copy(src_hbm.at[pl.ds(0, _CH)], sbufs[t],
                                  lds[t]).wait()
            pltpu.make_async_copy(dst_hbm.at[pl.ds(0, _CH)], dbufs[t],
                                  lds[t]).wait()

        def wait_sc(t):
            pltpu.make_async_copy(vbufs[t], acc.at[dbufs[t]], scs[t]).wait()

        def edge_pass(gather_fn):
            issue_pref(0, 0)
            issue_pref(1, 1)

            def quad(cc, _):
                for slot in range(4):
                    c = cc * 4 + slot
                    wait_ld(slot)
                    gather_fn(slot)
                    t2 = (slot + 2) % 4
                    if slot < 2:
                        @pl.when(cc > 0)
                        def _w():
                            wait_sc(t2)
                    else:
                        wait_sc(t2)

                    @pl.when(c + 2 < nch)
                    def _p():
                        issue_pref(c + 2, t2)

                    pltpu.async_copy(vbufs[slot], acc.at[dbufs[slot]],
                                     scs[slot], add=True)
                return _
            lax.fori_loop(0, nch // 4, quad, None)
            wait_sc((nch - 2) % 4)
            wait_sc((nch - 1) % 4)

        pltpu.sync_copy(consts_hbm, c_loc)

        def zfill(i, _):
            q_loc[pl.ds(i * 16, 16)] = jnp.zeros((16,), jnp.float32)
            return _
        lax.fori_loop(0, nv, zfill, None)
        pltpu.sync_copy(q_loc, acc.at[nsl])

        def ones_fill(slot):
            def ofill(i, _):
                vbufs[slot][pl.ds(i * 16, 16)] = jnp.full((16,), 1.0,
                                                          jnp.float32)
                return _
            lax.fori_loop(0, _CH // 16, ofill, None)
        for slot in range(4):
            ones_fill(slot)
        plsc.subcore_barrier()

        edge_pass(lambda slot: None)
        plsc.subcore_barrier()

        pltpu.sync_copy(acc.at[nsl], q_loc)
        pltpu.sync_copy(g_hbm.at[nsl], p_loc)
        t0 = c_loc[pl.ds(0, 16)]
        bfv = c_loc[pl.ds((k_hops + 1) * 16, 16)]

        def init_body(i, _):
            sl = pl.ds(i * 16, 16)
            dv = _newton_rsqrt(q_loc[sl] + 1.0)
            dinv_loc[sl] = dv
            gv = p_loc[sl]
            out_loc[sl] = t0 * gv + bfv
            p_loc[sl] = dv * gv
            return _
        lax.fori_loop(0, nv, init_body, None)

        pltpu.sync_copy(p_loc, p_sh.at[nsl])
        pltpu.sync_copy(p_loc, acc.at[nsl])
        plsc.subcore_barrier()

        def hop_gather(slot):
            pltpu.sync_copy(p_sh.at[sbufs[slot]], vbufs[slot])

        for k in range(1, k_hops + 1):
            edge_pass(hop_gather)
            plsc.subcore_barrier()

            pltpu.sync_copy(acc.at[nsl], q_loc)
            tk = c_loc[pl.ds(k * 16, 16)]

            def ew_body(i, _):
                sl = pl.ds(i * 16, 16)
                dv = dinv_loc[sl]
                y = dv * q_loc[sl]
                out_loc[sl] = out_loc[sl] + tk * y
                p_loc[sl] = dv * y
                return _
            lax.fori_loop(0, nv, ew_body, None)

            if k < k_hops:
                pltpu.sync_copy(p_loc, p_sh.at[nsl])
                pltpu.sync_copy(p_loc, acc.at[nsl])
                plsc.subcore_barrier()

        pltpu.sync_copy(out_loc, out_hbm.at[nsl])

    return pl.kernel(
        body,
        out_type=jax.ShapeDtypeStruct((n_pad,), jnp.float32),
        mesh=mesh,
        scratch_types=[
            pltpu.VMEM_SHARED((n_pad,), jnp.float32),
            pltpu.VMEM_SHARED((n_pad,), jnp.float32),
            pltpu.VMEM((slc,), jnp.float32),
            pltpu.VMEM((slc,), jnp.float32),
            pltpu.VMEM((slc,), jnp.float32),
            pltpu.VMEM((slc,), jnp.float32),
        ] + [pltpu.VMEM((_CH,), jnp.int32)] * 8
          + [pltpu.VMEM((_CH,), jnp.float32)] * 4
          + [pltpu.VMEM((16 * (k_hops + 2),), jnp.float32)]
          + [pltpu.SemaphoreType.DMA] * 8,
    )


def kernel(x, edge_index, W1, b1, W2, b2, temp, Wf, bf):
    n = x.shape[0]
    e = edge_index.shape[1]
    k_hops = temp.shape[0] - 1

    n_pad = ((n + _NT * 16 - 1) // (_NT * 16)) * (_NT * 16)
    e_pad = ((e + _NT * _CH * 4 - 1) // (_NT * _CH * 4)) * (_NT * _CH * 4)

    x_pad = jnp.pad(x, ((0, n_pad - n), (0, 0)))
    g = _run_mlp(x_pad, W1, b1, W2, b2, Wf, n_pad)[:, 0]

    src = jnp.pad(edge_index[0], (0, e_pad - e), constant_values=n_pad - 1)
    dst = jnp.pad(edge_index[1], (0, e_pad - e), constant_values=n_pad - 1)

    consts = jnp.concatenate([jnp.repeat(temp, 16), jnp.repeat(bf, 16)])

    prop = _make_propagate(n_pad, e_pad, k_hops)
    out = prop(src, dst, g, consts)
    return out[:n, None]


# speedup vs baseline: 121.6760x; 1.0445x over previous
import jax
import jax.numpy as jnp
from jax import lax
from jax.experimental import pallas as pl
from jax.experimental.pallas import tpu as pltpu
from jax.experimental.pallas import tpu_sc as plsc

_NT = 16
_CH = 5000


def _mlp_kernel(x_ref, w1_ref, b1_ref, w2_ref, b2_ref, wf_ref, g_ref):
    xb = x_ref[...]
    h1 = jnp.maximum(xb * w1_ref[...] + b1_ref[...][None, :], 0.0)
    h2 = jnp.dot(h1, w2_ref[...], preferred_element_type=jnp.float32)
    h2 = jnp.maximum(h2 + b2_ref[...][None, :], 0.0)
    g_ref[...] = jnp.sum(h2 * wf_ref[...][:, 0][None, :], axis=1,
                         keepdims=True)


def _run_mlp(x_pad, W1, b1, W2, b2, Wf, n_pad):
    blk = n_pad // 8
    full = lambda shape: pl.BlockSpec(shape, lambda i: (0,) * len(shape))
    return pl.pallas_call(
        _mlp_kernel,
        grid=(8,),
        in_specs=[
            pl.BlockSpec((blk, 1), lambda i: (i, 0)),
            full((1, 32)), full((32,)), full((32, 64)), full((64,)),
            full((64, 1)),
        ],
        out_specs=pl.BlockSpec((blk, 1), lambda i: (i, 0)),
        out_shape=jax.ShapeDtypeStruct((n_pad, 1), jnp.float32),
    )(x_pad, W1, b1, W2, b2, Wf)


def _newton_rsqrt(d):
    i = lax.bitcast_convert_type(d, jnp.int32)
    i = jnp.int32(0x5F3759DF) - lax.shift_right_arithmetic(i, 1)
    y = lax.bitcast_convert_type(i, jnp.float32)
    for _ in range(3):
        y = y * (1.5 - 0.5 * d * y * y)
    return y


def _make_propagate(n_pad, e_pad, k_hops, packed):
    slc = n_pad // _NT
    ec = e_pad // _NT
    nch = ec // _CH
    nv = slc // 16

    mesh = plsc.VectorSubcoreMesh(core_axis_name="c", subcore_axis_name="s",
                                  num_cores=1)

    def body(*refs):
        if packed:
            edges_hbm, g_hbm, consts_hbm, out_hbm, *rest = refs
            src_at = lambda e0: edges_hbm.at[pl.ds(e0, _CH)]
            dst_at = lambda e0: edges_hbm.at[pl.ds(e_pad + e0, _CH)]
        else:
            src_hbm, dst_hbm, g_hbm, consts_hbm, out_hbm, *rest = refs
            src_at = lambda e0: src_hbm.at[pl.ds(e0, _CH)]
            dst_at = lambda e0: dst_hbm.at[pl.ds(e0, _CH)]
        (p_sh, acc, p_loc, dinv_loc, out_loc, q_loc,
         sb0, sb1, sb2, sb3, db0, db1, db2, db3,
         vb0, vb1, vb2, vb3, c_loc,
         ld0, ld1, ld2, ld3, sc0, sc1, sc2, sc3) = rest
        sbufs = (sb0, sb1, sb2, sb3)
        dbufs = (db0, db1, db2, db3)
        vbufs = (vb0, vb1, vb2, vb3)
        lds = (ld0, ld1, ld2, ld3)
        scs = (sc0, sc1, sc2, sc3)

        s = lax.axis_index("s")
        base_n = s * slc
        base_e = s * ec
        nsl = pl.ds(base_n, slc)

        def issue_pref(chunk_idx, t):
            e0 = base_e + chunk_idx * _CH
            pltpu.async_copy(src_at(e0), sbufs[t], lds[t])
            pltpu.async_copy(dst_at(e0), dbufs[t], lds[t])

        def wait_ld(t):
            pltpu.make_async_copy(src_at(0), sbufs[t], lds[t]).wait()
            pltpu.make_async_copy(dst_at(0), dbufs[t], lds[t]).wait()

        def wait_sc(t):
            pltpu.make_async_copy(vbufs[t], acc.at[dbufs[t]], scs[t]).wait()

        def edge_pass(gather_fn):
            issue_pref(0, 0)
            issue_pref(1, 1)

            def quad(cc, _):
                for slot in range(4):
                    c = cc * 4 + slot
                    wait_ld(slot)
                    gather_fn(slot)
                    t2 = (slot + 2) % 4
                    if slot < 2:
                        @pl.when(cc > 0)
                        def _w():
                            wait_sc(t2)
                    else:
                        wait_sc(t2)

                    @pl.when(c + 2 < nch)
                    def _p():
                        issue_pref(c + 2, t2)

                    pltpu.async_copy(vbufs[slot], acc.at[dbufs[slot]],
                                     scs[slot], add=True)
                return _
            lax.fori_loop(0, nch // 4, quad, None)
            wait_sc((nch - 2) % 4)
            wait_sc((nch - 1) % 4)

        pltpu.sync_copy(consts_hbm, c_loc)

        def zfill(i, _):
            q_loc[pl.ds(i * 16, 16)] = jnp.zeros((16,), jnp.float32)
            return _
        lax.fori_loop(0, nv, zfill, None)
        pltpu.sync_copy(q_loc, acc.at[nsl])

        def ones_fill(slot):
            def ofill(i, _):
                vbufs[slot][pl.ds(i * 16, 16)] = jnp.full((16,), 1.0,
                                                          jnp.float32)
                return _
            lax.fori_loop(0, _CH // 16, ofill, None)
        for slot in range(4):
            ones_fill(slot)
        plsc.subcore_barrier()

        edge_pass(lambda slot: None)
        plsc.subcore_barrier()

        pltpu.sync_copy(acc.at[nsl], q_loc)
        pltpu.sync_copy(g_hbm.at[nsl], p_loc)
        t0 = c_loc[pl.ds(0, 16)]
        bfv = c_loc[pl.ds((k_hops + 1) * 16, 16)]

        def init_body(i, _):
            sl = pl.ds(i * 16, 16)
            dv = _newton_rsqrt(q_loc[sl] + 1.0)
            dinv_loc[sl] = dv
            gv = p_loc[sl]
            out_loc[sl] = t0 * gv + bfv
            p_loc[sl] = dv * gv
            return _
        lax.fori_loop(0, nv, init_body, None)

        pltpu.sync_copy(p_loc, p_sh.at[nsl])
        pltpu.sync_copy(p_loc, acc.at[nsl])
        plsc.subcore_barrier()

        def hop_gather(slot):
            pltpu.sync_copy(p_sh.at[sbufs[slot]], vbufs[slot])

        for k in range(1, k_hops + 1):
            edge_pass(hop_gather)
            plsc.subcore_barrier()

            pltpu.sync_copy(acc.at[nsl], q_loc)
            tk = c_loc[pl.ds(k * 16, 16)]

            def ew_body(i, _):
                sl = pl.ds(i * 16, 16)
                dv = dinv_loc[sl]
                y = dv * q_loc[sl]
                out_loc[sl] = out_loc[sl] + tk * y
                p_loc[sl] = dv * y
                return _
            lax.fori_loop(0, nv, ew_body, None)

            if k < k_hops:
                pltpu.sync_copy(p_loc, p_sh.at[nsl])
                pltpu.sync_copy(p_loc, acc.at[nsl])
                plsc.subcore_barrier()

        pltpu.sync_copy(out_loc, out_hbm.at[nsl])

    return pl.kernel(
        body,
        out_type=jax.ShapeDtypeStruct((n_pad,), jnp.float32),
        mesh=mesh,
        scratch_types=[
            pltpu.VMEM_SHARED((n_pad,), jnp.float32),
            pltpu.VMEM_SHARED((n_pad,), jnp.float32),
            pltpu.VMEM((slc,), jnp.float32),
            pltpu.VMEM((slc,), jnp.float32),
            pltpu.VMEM((slc,), jnp.float32),
            pltpu.VMEM((slc,), jnp.float32),
        ] + [pltpu.VMEM((_CH,), jnp.int32)] * 8
          + [pltpu.VMEM((_CH,), jnp.float32)] * 4
          + [pltpu.VMEM((16 * (k_hops + 2),), jnp.float32)]
          + [pltpu.SemaphoreType.DMA] * 8,
    )


def kernel(x, edge_index, W1, b1, W2, b2, temp, Wf, bf):
    n = x.shape[0]
    e = edge_index.shape[1]
    k_hops = temp.shape[0] - 1

    n_pad = ((n + _NT * 16 - 1) // (_NT * 16)) * (_NT * 16)
    e_pad = ((e + _NT * _CH * 4 - 1) // (_NT * _CH * 4)) * (_NT * _CH * 4)

    x_pad = jnp.pad(x, ((0, n_pad - n), (0, 0)))
    g = _run_mlp(x_pad, W1, b1, W2, b2, Wf, n_pad)[:, 0]

    consts = jnp.concatenate([jnp.repeat(temp, 16), jnp.repeat(bf, 16)])

    if e_pad == e:
        prop = _make_propagate(n_pad, e_pad, k_hops, packed=True)
        out = prop(edge_index.reshape(2 * e), g, consts)
    else:
        src = jnp.pad(edge_index[0], (0, e_pad - e), constant_values=n_pad - 1)
        dst = jnp.pad(edge_index[1], (0, e_pad - e), constant_values=n_pad - 1)
        prop = _make_propagate(n_pad, e_pad, k_hops, packed=False)
        out = prop(src, dst, g, consts)
    return out[:n, None]
